# Initial kernel scaffold; baseline (speedup 1.0000x reference)
import jax
import jax.numpy as jnp
from jax.experimental import pallas as pl


def kernel(x, z, edge_index, edge_attr, iso_type_2, iso_type_3, edge_index_2, edge_index_3, assignment_index_2, assignment_index_3, assignment2_to_subgraph, assignment3_to_subgraph, node_to_subgraph, subgraph_to_graph, z_table, conv6_wn, conv6_wr, conv6_b, conv7_wn, conv7_wr, conv7_b, conv4_wn, conv4_wr, conv4_b, conv5_wn, conv5_wr, conv5_b, conv1_w, conv1_wr, conv1_b, conv2_w, conv2_wr, conv2_b, conv3_w, conv3_wr, conv3_b, fc1_w, fc1_b, fc2_w, fc2_b, fc3_w, fc3_b):
    raise NotImplementedError("write your pallas kernel here")


# speedup vs baseline: 1.8611x; 1.8611x over previous
import functools

import jax
import jax.numpy as jnp
from jax import lax
from jax.experimental import pallas as pl
from jax.experimental.pallas import tpu as pltpu
import jax.experimental.pallas.tpu_sc as plsc

F32 = jnp.float32
NC, NS = 2, 16
NW = NC * NS
CH = 128
IB = 4
ALIGN = NS * CH * IB
BR = 512

_MESH = plsc.VectorSubcoreMesh(
    core_axis_name="c", subcore_axis_name="s", num_cores=NC, num_subcores=NS)


def _ru(x, m):
  return (x + m - 1) // m * m


@functools.lru_cache(None)
def _build_msg_pass(e_pad, n_pad, d):
  dh = d // 2
  ept = e_pad // NS
  ngrp = ept // (CH * IB)
  rpt = n_pad // NS

  @functools.partial(
      pl.kernel, mesh=_MESH,
      compiler_params=pltpu.CompilerParams(use_tc_tiling_on_sc=False),
      out_type=jax.ShapeDtypeStruct((n_pad, d), F32),
      scratch_types=[
          pltpu.VMEM((IB, CH), jnp.int32),
          pltpu.VMEM((IB, CH), jnp.int32),
          pltpu.VMEM((CH, dh), F32),
          pltpu.SemaphoreType.DMA,
          pltpu.VMEM_SHARED((n_pad, dh), F32),
      ])
  def k(src2, dst2, u0, u1, zer, out, sidx, didx, rows, sem, acc):
    c = lax.axis_index("c")
    s = lax.axis_index("s")
    pltpu.sync_copy(zer, acc.at[pl.ds(s * rpt, rpt)])
    plsc.subcore_barrier()

    def run(u):
      def grp(g, _):
        row0 = s * (ept // CH) + g * IB
        pltpu.sync_copy(src2.at[pl.ds(row0, IB)], sidx)
        pltpu.sync_copy(dst2.at[pl.ds(row0, IB)], didx)
        for j in range(IB):
          pltpu.async_copy(u.at[sidx.at[j]], rows, sem).wait()
          pltpu.sync_copy(rows, acc.at[didx.at[j]], add=True)
        return _
      lax.fori_loop(0, ngrp, grp, None)

    pl.when(c == 0)(lambda: run(u0))
    pl.when(c == 1)(lambda: run(u1))
    plsc.subcore_barrier()
    pltpu.sync_copy(acc.at[pl.ds(s * rpt, rpt)],
                    out.at[pl.ds(s * rpt, rpt), pl.ds(c * dh, dh)])

  return k


@functools.lru_cache(None)
def _build_seg_rows(e_pad, n_pad, d):
  dh = d // 2
  ept = e_pad // NS
  ngrp = ept // (CH * IB)
  rpt = n_pad // NS

  @functools.partial(
      pl.kernel, mesh=_MESH,
      compiler_params=pltpu.CompilerParams(use_tc_tiling_on_sc=False),
      out_type=jax.ShapeDtypeStruct((n_pad, d), F32),
      scratch_types=[
          pltpu.VMEM((IB, CH), jnp.int32),
          pltpu.VMEM((CH, dh), F32),
          pltpu.SemaphoreType.DMA,
          pltpu.VMEM_SHARED((n_pad, dh), F32),
      ])
  def k(dst2, m, zer, out, didx, rows, sem, acc):
    c = lax.axis_index("c")
    s = lax.axis_index("s")
    pltpu.sync_copy(zer, acc.at[pl.ds(s * rpt, rpt)])
    plsc.subcore_barrier()

    def grp(g, _):
      row0 = s * (ept // CH) + g * IB
      pltpu.sync_copy(dst2.at[pl.ds(row0, IB)], didx)
      for j in range(IB):
        pltpu.sync_copy(
            m.at[pl.ds((row0 + j) * CH, CH), pl.ds(c * dh, dh)], rows)
        pltpu.sync_copy(rows, acc.at[didx.at[j]], add=True)
      return _
    lax.fori_loop(0, ngrp, grp, None)

    plsc.subcore_barrier()
    pltpu.sync_copy(acc.at[pl.ds(s * rpt, rpt)],
                    out.at[pl.ds(s * rpt, rpt), pl.ds(c * dh, dh)])

  return k


@functools.lru_cache(None)
def _build_hist(e_pad, n_pad):
  d = 32
  dh = d // 2
  ept = e_pad // NS
  ngrp = ept // (CH * IB)
  rpt = n_pad // NS

  @functools.partial(
      pl.kernel, mesh=_MESH,
      compiler_params=pltpu.CompilerParams(use_tc_tiling_on_sc=False),
      out_type=jax.ShapeDtypeStruct((n_pad, d), F32),
      scratch_types=[
          pltpu.VMEM((IB, CH), jnp.int32),
          pltpu.VMEM((CH, dh), F32),
          pltpu.SemaphoreType.DMA,
          pltpu.VMEM_SHARED((n_pad, dh), F32),
      ])
  def k(dst2, ones_h, zer, out, didx, ones_v, sem, acc):
    c = lax.axis_index("c")
    s = lax.axis_index("s")
    pltpu.sync_copy(zer, acc.at[pl.ds(s * rpt, rpt)])
    pltpu.sync_copy(ones_h, ones_v)
    plsc.subcore_barrier()

    def grp(g, _):
      row0 = s * (ept // CH) + g * IB
      pltpu.sync_copy(dst2.at[pl.ds(row0, IB)], didx)
      for j in range(IB):
        pltpu.sync_copy(ones_v, acc.at[didx.at[j]], add=True)
      return _
    lax.fori_loop(0, ngrp, grp, None)

    plsc.subcore_barrier()
    pltpu.sync_copy(acc.at[pl.ds(s * rpt, rpt)],
                    out.at[pl.ds(s * rpt, rpt), pl.ds(c * dh, dh)])

  return k


@functools.lru_cache(None)
def _build_gather(n_tab, b_pad, d):
  ibg = 2
  bpw = b_pad // NW
  ngrp = bpw // (CH * ibg)

  @functools.partial(
      pl.kernel, mesh=_MESH,
      compiler_params=pltpu.CompilerParams(use_tc_tiling_on_sc=False),
      out_type=jax.ShapeDtypeStruct((b_pad, d), F32),
      scratch_types=[
          pltpu.VMEM((ibg, CH), jnp.int32),
          pltpu.VMEM((CH, d), F32),
          pltpu.SemaphoreType.DMA,
      ])
  def k(tab, idx2, out, idxv, rows, sem):
    c = lax.axis_index("c")
    s = lax.axis_index("s")
    w = s * NC + c

    def grp(g, _):
      row0 = w * (bpw // CH) + g * ibg
      pltpu.sync_copy(idx2.at[pl.ds(row0, ibg)], idxv)
      for j in range(ibg):
        pltpu.async_copy(tab.at[idxv.at[j]], rows, sem).wait()
        pltpu.sync_copy(rows, out.at[pl.ds((row0 + j) * CH, CH)])
      return _
    lax.fori_loop(0, ngrp, grp, None)

  return k


def _pad_idx(idx, pad_val, length):
  idx = idx.astype(jnp.int32)
  n = idx.shape[0]
  out = jnp.full((length,), pad_val, jnp.int32).at[:n].set(idx)
  return out.reshape(length // CH, CH)


def _msg_pass(u, src, dst, n, n_pad):
  e_pad = _ru(src.shape[0], ALIGN)
  d = u.shape[1]
  dh = d // 2
  src2 = _pad_idx(src, 0, e_pad)
  dst2 = _pad_idx(dst, n, e_pad)
  zer = jnp.zeros((n_pad // NS, dh), F32)
  k = _build_msg_pass(e_pad, n_pad, d)
  return k(src2, dst2, u[:, :dh], u[:, dh:], zer)


def _seg_rows(m, dst, n, n_pad):
  e = m.shape[0]
  e_pad = _ru(e, ALIGN)
  d = m.shape[1]
  if e_pad != e:
    m = jnp.zeros((e_pad, d), F32).at[:e].set(m)
  dst2 = _pad_idx(dst, n, e_pad)
  zer = jnp.zeros((n_pad // NS, d // 2), F32)
  return _build_seg_rows(e_pad, n_pad, d)(dst2, m, zer)


def _hist(dst, n, n_pad):
  e_pad = _ru(dst.shape[0], ALIGN)
  dst2 = _pad_idx(dst, n, e_pad)
  ones_h = jnp.ones((CH, 16), F32)
  zer = jnp.zeros((n_pad // NS, 16), F32)
  return _build_hist(e_pad, n_pad)(dst2, ones_h, zer)


def _gather(tab, idx):
  b = idx.shape[0]
  b_pad = _ru(b, ALIGN)
  idx2 = _pad_idx(idx, 0, b_pad)
  return _build_gather(tab.shape[0], b_pad, tab.shape[1])(tab, idx2)


def _elu(x):
  return jnp.where(x > 0, x, jnp.exp(jnp.minimum(x, 0.0)) - 1.0)


@functools.lru_cache(None)
def _build_matmul(rows, kdim, dout, has_add, act):
  def body(*refs):
    if has_add:
      x, w, b, a, o = refs
      acc = jnp.dot(x[...], w[...], preferred_element_type=F32) + b[...] + a[...]
    else:
      x, w, b, o = refs
      acc = jnp.dot(x[...], w[...], preferred_element_type=F32) + b[...]
    o[...] = _elu(acc) if act else acc

  in_specs = [
      pl.BlockSpec((BR, kdim), lambda i: (i, 0)),
      pl.BlockSpec((kdim, dout), lambda i: (0, 0)),
      pl.BlockSpec((1, dout), lambda i: (0, 0)),
  ]
  if has_add:
    in_specs.append(pl.BlockSpec((BR, dout), lambda i: (i, 0)))
  return pl.pallas_call(
      body,
      grid=(rows // BR,),
      in_specs=in_specs,
      out_specs=pl.BlockSpec((BR, dout), lambda i: (i, 0)),
      out_shape=jax.ShapeDtypeStruct((rows, dout), F32),
  )


def _matmul(x, w, b=None, addend=None, act=False):
  rows, kdim = x.shape
  dout = w.shape[1]
  b = jnp.zeros((1, dout), F32) if b is None else b.reshape(1, dout)
  k = _build_matmul(rows, kdim, dout, addend is not None, act)
  if addend is not None:
    return k(x, w, b, addend)
  return k(x, w, b)


@functools.lru_cache(None)
def _build_combine(rows, dout, act, nin):
  def body(a, b, bias, o):
    o[...] = _elu(a[...] + b[...] + bias[...])

  return pl.pallas_call(
      body,
      grid=(rows // BR,),
      in_specs=[
          pl.BlockSpec((BR, dout), lambda i: (i, 0)),
          pl.BlockSpec((BR, dout), lambda i: (i, 0)),
          pl.BlockSpec((1, dout), lambda i: (0, 0)),
      ],
      out_specs=pl.BlockSpec((BR, dout), lambda i: (i, 0)),
      out_shape=jax.ShapeDtypeStruct((rows, dout), F32),
  )


def _combine_elu(a, b, bias=None):
  rows, dout = a.shape
  bias = jnp.zeros((1, dout), F32) if bias is None else bias.reshape(1, dout)
  return _build_combine(rows, dout, True, 2)(a, b, bias)


@functools.lru_cache(None)
def _build_rgcn_combine(rows, dout):
  def body(g, ea, o):
    gg = g[...]
    ee = ea[...]
    acc = ee[:, 0:1] * gg[:, 0 * dout:1 * dout]
    for r in range(1, 4):
      acc = acc + ee[:, r:r + 1] * gg[:, r * dout:(r + 1) * dout]
    o[...] = acc

  return pl.pallas_call(
      body,
      grid=(rows // BR,),
      in_specs=[
          pl.BlockSpec((BR, 4 * dout), lambda i: (i, 0)),
          pl.BlockSpec((BR, 8), lambda i: (i, 0)),
      ],
      out_specs=pl.BlockSpec((BR, dout), lambda i: (i, 0)),
      out_shape=jax.ShapeDtypeStruct((rows, dout), F32),
  )


def _rgcn_combine(g, ea, dout):
  return _build_rgcn_combine(g.shape[0], dout)(g, ea)


@functools.lru_cache(None)
def _build_rowdiv(rows, d):
  def body(x, c, o):
    o[...] = x[...] / jnp.maximum(c[:, 0:1], 1.0)

  return pl.pallas_call(
      body,
      grid=(rows // BR,),
      in_specs=[
          pl.BlockSpec((BR, d), lambda i: (i, 0)),
          pl.BlockSpec((BR, 32), lambda i: (i, 0)),
      ],
      out_specs=pl.BlockSpec((BR, d), lambda i: (i, 0)),
      out_shape=jax.ShapeDtypeStruct((rows, d), F32),
  )


def _rowdiv(x, cnt):
  return _build_rowdiv(x.shape[0], x.shape[1])(x, cnt)


def _pad_rows(x, rows):
  return jnp.zeros((rows, x.shape[1]), F32).at[:x.shape[0]].set(x)


def kernel(x, z, edge_index, edge_attr, iso_type_2, iso_type_3, edge_index_2,
           edge_index_3, assignment_index_2, assignment_index_3,
           assignment2_to_subgraph, assignment3_to_subgraph, node_to_subgraph,
           subgraph_to_graph, z_table, conv6_wn, conv6_wr, conv6_b, conv7_wn,
           conv7_wr, conv7_b, conv4_wn, conv4_wr, conv4_b, conv5_wn, conv5_wr,
           conv5_b, conv1_w, conv1_wr, conv1_b, conv2_w, conv2_wr, conv2_b,
           conv3_w, conv3_wr, conv3_b, fc1_w, fc1_b, fc2_w, fc2_b, fc3_w,
           fc3_b):
  n1 = x.shape[0]
  n2 = iso_type_2.shape[0]
  n3 = iso_type_3.shape[0]
  s_num = subgraph_to_graph.shape[0]
  g_num = 256
  n1p = _ru(n1 + 1, BR)
  n2p = _ru(n2 + 1, BR)
  n3p = _ru(n3 + 1, BR)
  sp = _ru(s_num + 1, BR)
  gp = _ru(g_num + 1, BR)

  src1, dst1 = edge_index[0], edge_index[1]
  src2, dst2 = edge_index_2[0], edge_index_2[1]
  src3, dst3 = edge_index_3[0], edge_index_3[1]

  u6 = _pad_rows(_gather(conv6_wn, iso_type_3)[:n3], n3p)
  r6 = _pad_rows(_gather(conv6_wr, iso_type_3)[:n3], n3p)
  agg6 = _msg_pass(u6, src3, dst3, n3, n3p)
  h6 = _combine_elu(agg6, r6, conv6_b)

  u7 = _matmul(h6, conv7_wn)
  agg7 = _msg_pass(u7, src3, dst3, n3, n3p)
  h7 = _matmul(h6, conv7_wr, conv7_b, addend=agg7, act=True)

  cnt_s3 = _hist(assignment3_to_subgraph, s_num, sp)
  x_3 = _rowdiv(_seg_rows(h7, assignment3_to_subgraph, s_num, sp), cnt_s3)

  cnt_a3 = _hist(assignment_index_3[0], n2, n2p)
  h_a3 = _rowdiv(
      _msg_pass(h7, assignment_index_3[1], assignment_index_3[0], n2, n2p),
      cnt_a3)

  e4n = _pad_rows(_gather(conv4_wn[64:], iso_type_2)[:n2], n2p)
  e4r = _pad_rows(_gather(conv4_wr[64:], iso_type_2)[:n2], n2p)
  u4 = _matmul(h_a3, conv4_wn[:64], addend=e4n)
  r4 = _matmul(h_a3, conv4_wr[:64], conv4_b, addend=e4r)
  agg4 = _msg_pass(u4, src2, dst2, n2, n2p)
  h4 = _combine_elu(agg4, r4)

  u5 = _matmul(h4, conv5_wn)
  agg5 = _msg_pass(u5, src2, dst2, n2, n2p)
  h5 = _matmul(h4, conv5_wr, conv5_b, addend=agg5, act=True)

  cnt_s2 = _hist(assignment2_to_subgraph, s_num, sp)
  x_2 = _rowdiv(_seg_rows(h5, assignment2_to_subgraph, s_num, sp), cnt_s2)

  cnt_a2 = _hist(assignment_index_2[0], n1, n1p)
  h_a2 = _rowdiv(
      _msg_pass(h5, assignment_index_2[1], assignment_index_2[0], n1, n1p),
      cnt_a2)

  zt = jnp.zeros((z_table.shape[0], 16), F32).at[:, :8].set(z_table)
  z_emb = _gather(zt, z)[:n1, :8]
  x1 = _pad_rows(jnp.concatenate([z_emb, x, h_a2[:n1]], axis=1), n1p)

  ea = jnp.zeros((_ru(edge_attr.shape[0], ALIGN), 8), F32)
  ea = ea.at[:edge_attr.shape[0], :4].set(edge_attr)

  def rgcn(h, w, wr, b, dout):
    wflat = jnp.transpose(w, (1, 0, 2)).reshape(w.shape[1], 4 * dout)
    u = _matmul(h, wflat)
    r = _matmul(h, wr, b)
    g = _gather(u, src1)
    m = _rgcn_combine(g, ea, dout)
    agg = _seg_rows(m, dst1, n1, n1p)
    return _combine_elu(agg, r)

  h1 = rgcn(x1, conv1_w, conv1_wr, conv1_b, 32)
  h2 = rgcn(h1, conv2_w, conv2_wr, conv2_b, 64)
  h3 = rgcn(h2, conv3_w, conv3_wr, conv3_b, 64)

  cnt_s1 = _hist(node_to_subgraph, s_num, sp)
  x_1 = _rowdiv(_seg_rows(h3, node_to_subgraph, s_num, sp), cnt_s1)

  xc = jnp.concatenate([x_1[:s_num], x_2[:s_num], x_3[:s_num]], axis=1)
  cnt_g = _hist(subgraph_to_graph, g_num, gp)
  gsum = _seg_rows(xc, subgraph_to_graph, g_num, gp)
  g0 = _rowdiv(gsum, cnt_g)

  g1 = _matmul(g0, fc1_w, fc1_b, act=True)
  g2 = _matmul(g1, fc2_w, fc2_b, act=True)
  fc3w = jnp.zeros((fc3_w.shape[0], 8), F32).at[:, :1].set(fc3_w)
  fc3b = jnp.zeros((8,), F32).at[:1].set(fc3_b)
  g3 = _matmul(g2, fc3w, fc3b)
  return g3[:g_num, :1]


# speedup vs baseline: 2.6032x; 1.3987x over previous
import functools

import jax
import jax.numpy as jnp
from jax import lax
from jax.experimental import pallas as pl
from jax.experimental.pallas import tpu as pltpu
import jax.experimental.pallas.tpu_sc as plsc

F32 = jnp.float32
NC, NS = 2, 16
NW = NC * NS
CH = 128
IB = 4
ALIGN = NS * CH * IB
BR = 512

_MESH = plsc.VectorSubcoreMesh(
    core_axis_name="c", subcore_axis_name="s", num_cores=NC, num_subcores=NS)


def _ru(x, m):
  return (x + m - 1) // m * m


PAD = 512


@functools.lru_cache(None)
def _build_msg_pass(e_pad, n_pad, d):
  dh = d // 2
  ngrp = e_pad // (CH * IB)
  nloop = (ngrp + NS - 1) // NS
  rpt = n_pad // NS

  @functools.partial(
      pl.kernel, mesh=_MESH,
      compiler_params=pltpu.CompilerParams(use_tc_tiling_on_sc=False),
      out_type=jax.ShapeDtypeStruct((n_pad, d), F32),
      scratch_types=[
          pltpu.VMEM((IB, CH), jnp.int32),
          pltpu.VMEM((IB, CH), jnp.int32),
          pltpu.VMEM((IB, CH, dh), F32),
          pltpu.SemaphoreType.DMA,
          pltpu.SemaphoreType.DMA,
          pltpu.VMEM_SHARED((n_pad, dh), F32),
      ])
  def k(src2, dst2, u0, u1, zer, out, sidx, didx, rows, gsem, ssem, acc):
    c = lax.axis_index("c")
    s = lax.axis_index("s")
    pltpu.sync_copy(zer, acc.at[pl.ds(s * rpt, rpt)])
    plsc.subcore_barrier()

    def run(u):
      def grp(g, _):
        gid = g * NS + s

        @pl.when(gid < ngrp)
        def _do():
          row0 = gid * IB
          pltpu.sync_copy(src2.at[pl.ds(row0, IB)], sidx)
          pltpu.sync_copy(dst2.at[pl.ds(row0, IB)], didx)
          gds = [pltpu.async_copy(u.at[sidx.at[j]], rows.at[j], gsem)
                 for j in range(IB)]
          for d_ in gds:
            d_.wait()
          sds = [pltpu.async_copy(rows.at[j], acc.at[didx.at[j]], ssem,
                                  add=True) for j in range(IB)]
          for d_ in sds:
            d_.wait()
        return _
      lax.fori_loop(0, nloop, grp, None)

    pl.when(c == 0)(lambda: run(u0))
    pl.when(c == 1)(lambda: run(u1))
    plsc.subcore_barrier()
    pltpu.sync_copy(acc.at[pl.ds(s * rpt, rpt)],
                    out.at[pl.ds(s * rpt, rpt), pl.ds(c * dh, dh)])

  return k


@functools.lru_cache(None)
def _build_seg_rows(e_pad, n_pad, d):
  dh = d // 2
  ngrp = e_pad // (CH * IB)
  nloop = (ngrp + NS - 1) // NS
  rpt = n_pad // NS

  @functools.partial(
      pl.kernel, mesh=_MESH,
      compiler_params=pltpu.CompilerParams(use_tc_tiling_on_sc=False),
      out_type=jax.ShapeDtypeStruct((n_pad, d), F32),
      scratch_types=[
          pltpu.VMEM((IB, CH), jnp.int32),
          pltpu.VMEM((IB, CH, dh), F32),
          pltpu.SemaphoreType.DMA,
          pltpu.SemaphoreType.DMA,
          pltpu.VMEM_SHARED((n_pad, dh), F32),
      ])
  def k(dst2, m, zer, out, didx, rows, lsem, ssem, acc):
    c = lax.axis_index("c")
    s = lax.axis_index("s")
    pltpu.sync_copy(zer, acc.at[pl.ds(s * rpt, rpt)])
    plsc.subcore_barrier()

    def grp(g, _):
      gid = g * NS + s

      @pl.when(gid < ngrp)
      def _do():
        row0 = gid * IB
        pltpu.sync_copy(dst2.at[pl.ds(row0, IB)], didx)
        lds = [pltpu.async_copy(
            m.at[pl.ds((row0 + j) * CH, CH), pl.ds(c * dh, dh)],
            rows.at[j], lsem) for j in range(IB)]
        for d_ in lds:
          d_.wait()
        sds = [pltpu.async_copy(rows.at[j], acc.at[didx.at[j]], ssem,
                                add=True) for j in range(IB)]
        for d_ in sds:
          d_.wait()
      return _
    lax.fori_loop(0, nloop, grp, None)

    plsc.subcore_barrier()
    pltpu.sync_copy(acc.at[pl.ds(s * rpt, rpt)],
                    out.at[pl.ds(s * rpt, rpt), pl.ds(c * dh, dh)])

  return k


@functools.lru_cache(None)
def _build_hist(e_pad, n_pad):
  d = 32
  dh = d // 2
  ngrp = e_pad // (CH * IB)
  nloop = (ngrp + NS - 1) // NS
  rpt = n_pad // NS

  @functools.partial(
      pl.kernel, mesh=_MESH,
      compiler_params=pltpu.CompilerParams(use_tc_tiling_on_sc=False),
      out_type=jax.ShapeDtypeStruct((n_pad, d), F32),
      scratch_types=[
          pltpu.VMEM((IB, CH), jnp.int32),
          pltpu.VMEM((CH, dh), F32),
          pltpu.SemaphoreType.DMA,
          pltpu.SemaphoreType.DMA,
          pltpu.VMEM_SHARED((n_pad, dh), F32),
      ])
  def k(dst2, ones_h, zer, out, didx, ones_v, isem, ssem, acc):
    c = lax.axis_index("c")
    s = lax.axis_index("s")
    pltpu.sync_copy(zer, acc.at[pl.ds(s * rpt, rpt)])
    pltpu.sync_copy(ones_h, ones_v)
    plsc.subcore_barrier()

    def grp(g, _):
      gid = g * NS + s

      @pl.when(gid < ngrp)
      def _do():
        row0 = gid * IB
        pltpu.sync_copy(dst2.at[pl.ds(row0, IB)], didx)
        sds = [pltpu.async_copy(ones_v, acc.at[didx.at[j]], ssem, add=True)
               for j in range(IB)]
        for d_ in sds:
          d_.wait()
      return _
    lax.fori_loop(0, nloop, grp, None)

    plsc.subcore_barrier()
    pltpu.sync_copy(acc.at[pl.ds(s * rpt, rpt)],
                    out.at[pl.ds(s * rpt, rpt), pl.ds(c * dh, dh)])

  return k


@functools.lru_cache(None)
def _build_gather(n_tab, b_pad, d):
  ibg = 2
  ngrp = b_pad // (CH * ibg)
  nloop = (ngrp + NW - 1) // NW
  tc_tiling = (d % 128 == 0)

  @functools.partial(
      pl.kernel, mesh=_MESH,
      compiler_params=pltpu.CompilerParams(use_tc_tiling_on_sc=tc_tiling),
      out_type=jax.ShapeDtypeStruct((b_pad, d), F32),
      scratch_types=[
          pltpu.VMEM((ibg, CH), jnp.int32),
          pltpu.VMEM((ibg, CH, d), F32),
          pltpu.SemaphoreType.DMA,
          pltpu.SemaphoreType.DMA,
      ])
  def k(tab, idx2, out, idxv, rows, gsem, wsem):
    c = lax.axis_index("c")
    s = lax.axis_index("s")
    w = s * NC + c

    def grp(g, _):
      gid = g * NW + w

      @pl.when(gid < ngrp)
      def _do():
        row0 = gid * ibg
        pltpu.sync_copy(idx2.at[pl.ds(row0, ibg)], idxv)
        gds = [pltpu.async_copy(tab.at[idxv.at[j]], rows.at[j], gsem)
               for j in range(ibg)]
        for d_ in gds:
          d_.wait()
        wds = [pltpu.async_copy(
            rows.at[j], out.at[pl.ds((row0 + j) * CH, CH)], wsem)
            for j in range(ibg)]
        for d_ in wds:
          d_.wait()
      return _
    lax.fori_loop(0, nloop, grp, None)

  return k


def _pad_idx(idx, pad_val, length):
  idx = idx.astype(jnp.int32)
  n = idx.shape[0]
  out = jnp.full((length,), pad_val, jnp.int32).at[:n].set(idx)
  return out.reshape(length // CH, CH)


def _msg_pass(u, src, dst, n, n_pad):
  e_pad = _ru(src.shape[0], PAD)
  d = u.shape[1]
  dh = d // 2
  src2 = _pad_idx(src, 0, e_pad)
  dst2 = _pad_idx(dst, n, e_pad)
  zer = jnp.zeros((n_pad // NS, dh), F32)
  k = _build_msg_pass(e_pad, n_pad, d)
  return k(src2, dst2, u[:, :dh], u[:, dh:], zer)


def _seg_rows(m, dst, n, n_pad):
  e = m.shape[0]
  e_pad = _ru(e, PAD)
  d = m.shape[1]
  if e_pad != e:
    m = jnp.zeros((e_pad, d), F32).at[:e].set(m)
  dst2 = _pad_idx(dst, n, e_pad)
  zer = jnp.zeros((n_pad // NS, d // 2), F32)
  return _build_seg_rows(e_pad, n_pad, d)(dst2, m, zer)


def _hist(dst, n, n_pad):
  e_pad = _ru(dst.shape[0], PAD)
  dst2 = _pad_idx(dst, n, e_pad)
  ones_h = jnp.ones((CH, 16), F32)
  zer = jnp.zeros((n_pad // NS, 16), F32)
  return _build_hist(e_pad, n_pad)(dst2, ones_h, zer)


def _gather(tab, idx):
  b = idx.shape[0]
  b_pad = _ru(b, PAD)
  idx2 = _pad_idx(idx, 0, b_pad)
  return _build_gather(tab.shape[0], b_pad, tab.shape[1])(tab, idx2)


def _elu(x):
  return jnp.where(x > 0, x, jnp.exp(jnp.minimum(x, 0.0)) - 1.0)


@functools.lru_cache(None)
def _build_matmul(rows, kdim, dout, has_add, act):
  def body(*refs):
    if has_add:
      x, w, b, a, o = refs
      acc = jnp.dot(x[...], w[...], preferred_element_type=F32) + b[...] + a[...]
    else:
      x, w, b, o = refs
      acc = jnp.dot(x[...], w[...], preferred_element_type=F32) + b[...]
    o[...] = _elu(acc) if act else acc

  in_specs = [
      pl.BlockSpec((BR, kdim), lambda i: (i, 0)),
      pl.BlockSpec((kdim, dout), lambda i: (0, 0)),
      pl.BlockSpec((1, dout), lambda i: (0, 0)),
  ]
  if has_add:
    in_specs.append(pl.BlockSpec((BR, dout), lambda i: (i, 0)))
  return pl.pallas_call(
      body,
      grid=(rows // BR,),
      in_specs=in_specs,
      out_specs=pl.BlockSpec((BR, dout), lambda i: (i, 0)),
      out_shape=jax.ShapeDtypeStruct((rows, dout), F32),
  )


def _matmul(x, w, b=None, addend=None, act=False):
  rows, kdim = x.shape
  dout = w.shape[1]
  b = jnp.zeros((1, dout), F32) if b is None else b.reshape(1, dout)
  k = _build_matmul(rows, kdim, dout, addend is not None, act)
  if addend is not None:
    return k(x, w, b, addend)
  return k(x, w, b)


@functools.lru_cache(None)
def _build_combine(rows, dout, act, nin):
  def body(a, b, bias, o):
    o[...] = _elu(a[...] + b[...] + bias[...])

  return pl.pallas_call(
      body,
      grid=(rows // BR,),
      in_specs=[
          pl.BlockSpec((BR, dout), lambda i: (i, 0)),
          pl.BlockSpec((BR, dout), lambda i: (i, 0)),
          pl.BlockSpec((1, dout), lambda i: (0, 0)),
      ],
      out_specs=pl.BlockSpec((BR, dout), lambda i: (i, 0)),
      out_shape=jax.ShapeDtypeStruct((rows, dout), F32),
  )


def _combine_elu(a, b, bias=None):
  rows, dout = a.shape
  bias = jnp.zeros((1, dout), F32) if bias is None else bias.reshape(1, dout)
  return _build_combine(rows, dout, True, 2)(a, b, bias)


@functools.lru_cache(None)
def _build_rgcn_combine(rows, dout):
  def body(g, ea, o):
    gg = g[...]
    ee = ea[...]
    acc = ee[:, 0:1] * gg[:, 0 * dout:1 * dout]
    for r in range(1, 4):
      acc = acc + ee[:, r:r + 1] * gg[:, r * dout:(r + 1) * dout]
    o[...] = acc

  return pl.pallas_call(
      body,
      grid=(rows // BR,),
      in_specs=[
          pl.BlockSpec((BR, 4 * dout), lambda i: (i, 0)),
          pl.BlockSpec((BR, 8), lambda i: (i, 0)),
      ],
      out_specs=pl.BlockSpec((BR, dout), lambda i: (i, 0)),
      out_shape=jax.ShapeDtypeStruct((rows, dout), F32),
  )


def _rgcn_combine(g, ea, dout):
  return _build_rgcn_combine(g.shape[0], dout)(g, ea)


@functools.lru_cache(None)
def _build_rowdiv(rows, d):
  def body(x, c, o):
    o[...] = x[...] / jnp.maximum(c[:, 0:1], 1.0)

  return pl.pallas_call(
      body,
      grid=(rows // BR,),
      in_specs=[
          pl.BlockSpec((BR, d), lambda i: (i, 0)),
          pl.BlockSpec((BR, 32), lambda i: (i, 0)),
      ],
      out_specs=pl.BlockSpec((BR, d), lambda i: (i, 0)),
      out_shape=jax.ShapeDtypeStruct((rows, d), F32),
  )


def _rowdiv(x, cnt):
  return _build_rowdiv(x.shape[0], x.shape[1])(x, cnt)


def _pad_rows(x, rows):
  return jnp.zeros((rows, x.shape[1]), F32).at[:x.shape[0]].set(x)


def kernel(x, z, edge_index, edge_attr, iso_type_2, iso_type_3, edge_index_2,
           edge_index_3, assignment_index_2, assignment_index_3,
           assignment2_to_subgraph, assignment3_to_subgraph, node_to_subgraph,
           subgraph_to_graph, z_table, conv6_wn, conv6_wr, conv6_b, conv7_wn,
           conv7_wr, conv7_b, conv4_wn, conv4_wr, conv4_b, conv5_wn, conv5_wr,
           conv5_b, conv1_w, conv1_wr, conv1_b, conv2_w, conv2_wr, conv2_b,
           conv3_w, conv3_wr, conv3_b, fc1_w, fc1_b, fc2_w, fc2_b, fc3_w,
           fc3_b):
  n1 = x.shape[0]
  n2 = iso_type_2.shape[0]
  n3 = iso_type_3.shape[0]
  s_num = subgraph_to_graph.shape[0]
  g_num = 256
  n1p = _ru(n1 + 1, PAD)
  n2p = _ru(n2 + 1, PAD)
  n3p = _ru(n3 + 1, PAD)
  sp = _ru(s_num + 1, PAD)
  gp = _ru(g_num + 1, PAD)

  src1, dst1 = edge_index[0], edge_index[1]
  src2, dst2 = edge_index_2[0], edge_index_2[1]
  src3, dst3 = edge_index_3[0], edge_index_3[1]

  u6 = _gather(conv6_wn, iso_type_3)
  r6 = _gather(conv6_wr, iso_type_3)
  agg6 = _msg_pass(u6, src3, dst3, n3, n3p)
  h6 = _combine_elu(agg6, r6, conv6_b)

  u7 = _matmul(h6, conv7_wn)
  agg7 = _msg_pass(u7, src3, dst3, n3, n3p)
  h7 = _matmul(h6, conv7_wr, conv7_b, addend=agg7, act=True)

  cnt_s3 = _hist(assignment3_to_subgraph, s_num, sp)
  x_3 = _rowdiv(_seg_rows(h7, assignment3_to_subgraph, s_num, sp), cnt_s3)

  cnt_a3 = _hist(assignment_index_3[0], n2, n2p)
  h_a3 = _rowdiv(
      _msg_pass(h7, assignment_index_3[1], assignment_index_3[0], n2, n2p),
      cnt_a3)

  e4n = _gather(conv4_wn[64:], iso_type_2)
  e4r = _gather(conv4_wr[64:], iso_type_2)
  u4 = _matmul(h_a3, conv4_wn[:64], addend=e4n)
  r4 = _matmul(h_a3, conv4_wr[:64], conv4_b, addend=e4r)
  agg4 = _msg_pass(u4, src2, dst2, n2, n2p)
  h4 = _combine_elu(agg4, r4)

  u5 = _matmul(h4, conv5_wn)
  agg5 = _msg_pass(u5, src2, dst2, n2, n2p)
  h5 = _matmul(h4, conv5_wr, conv5_b, addend=agg5, act=True)

  cnt_s2 = _hist(assignment2_to_subgraph, s_num, sp)
  x_2 = _rowdiv(_seg_rows(h5, assignment2_to_subgraph, s_num, sp), cnt_s2)

  cnt_a2 = _hist(assignment_index_2[0], n1, n1p)
  h_a2 = _rowdiv(
      _msg_pass(h5, assignment_index_2[1], assignment_index_2[0], n1, n1p),
      cnt_a2)

  zt = jnp.zeros((z_table.shape[0], 16), F32).at[:, :8].set(z_table)
  z_emb = _gather(zt, z)[:, :8]
  x1 = jnp.concatenate([z_emb, _pad_rows(x, n1p), h_a2], axis=1)

  ea = jnp.zeros((_ru(edge_attr.shape[0], PAD), 8), F32)
  ea = ea.at[:edge_attr.shape[0], :4].set(edge_attr)

  def rgcn(h, w, wr, b, dout):
    wflat = jnp.transpose(w, (1, 0, 2)).reshape(w.shape[1], 4 * dout)
    u = _matmul(h, wflat)
    r = _matmul(h, wr, b)
    g = _gather(u, src1)
    m = _rgcn_combine(g, ea, dout)
    agg = _seg_rows(m, dst1, n1, n1p)
    return _combine_elu(agg, r)

  h1 = rgcn(x1, conv1_w, conv1_wr, conv1_b, 32)
  h2 = rgcn(h1, conv2_w, conv2_wr, conv2_b, 64)
  h3 = rgcn(h2, conv3_w, conv3_wr, conv3_b, 64)

  cnt_s1 = _hist(node_to_subgraph, s_num, sp)
  x_1 = _rowdiv(_seg_rows(h3, node_to_subgraph, s_num, sp), cnt_s1)

  xc = jnp.concatenate([x_1, x_2, x_3], axis=1)
  cnt_g = _hist(subgraph_to_graph, g_num, gp)
  gsum = _seg_rows(xc, subgraph_to_graph, g_num, gp)
  g0 = _rowdiv(gsum, cnt_g)

  g1 = _matmul(g0, fc1_w, fc1_b, act=True)
  g2 = _matmul(g1, fc2_w, fc2_b, act=True)
  fc3w = jnp.zeros((fc3_w.shape[0], 8), F32).at[:, :1].set(fc3_w)
  fc3b = jnp.zeros((8,), F32).at[:1].set(fc3_b)
  g3 = _matmul(g2, fc3w, fc3b)
  return g3[:g_num, :1]
